# baseline (speedup 1.0000x reference)
import jax
import jax.numpy as jnp
from jax.experimental import pallas as pl
from jax.experimental.pallas import tpu as pltpu

W = 256
H = 16
M = 144
ROWS = 529
IB = 8
SL = H * M * M // 128
NB = 4
NSTEP = W // NB


def _gather_body(rpi_ref, btT_ref, out_ref):
    iota = jax.lax.broadcasted_iota(jnp.int32, (ROWS, M), 0)
    btT = btT_ref[...]
    for rr in range(IB):
        onehot = (rpi_ref[rr:rr + 1, :] == iota).astype(jnp.float32)
        out_ref[:, rr, :] = jnp.dot(btT, onehot,
                                    preferred_element_type=jnp.float32)


def _add_body(bias_ref, att_hbm, out_hbm, in_buf, out_buf, in_sem, out_sem):
    i = pl.program_id(0)
    slot = jax.lax.rem(i, 2)
    nslot = 1 - slot

    def cp_in(step, sl):
        return pltpu.make_async_copy(
            att_hbm.at[pl.ds(step * NB, NB)], in_buf.at[sl], in_sem.at[sl])

    def cp_out(step, sl):
        return pltpu.make_async_copy(
            out_buf.at[sl], out_hbm.at[pl.ds(step * NB, NB)], out_sem.at[sl])

    @pl.when(i == 0)
    def _():
        cp_in(i, slot).start()

    @pl.when(i + 1 < NSTEP)
    def _():
        cp_in(i + 1, nslot).start()

    cp_in(i, slot).wait()

    @pl.when(i >= 2)
    def _():
        cp_out(i - 2, slot).wait()

    out_buf[slot] = in_buf[slot] + bias_ref[...][None]
    cp_out(i, slot).start()

    @pl.when(i == NSTEP - 1)
    def _():
        cp_out(i - 1, nslot).wait()
        cp_out(i, slot).wait()


def kernel(att_scores, bias_table, relative_position_index):
    bias = pl.pallas_call(
        _gather_body,
        grid=(M // IB,),
        in_specs=[
            pl.BlockSpec((IB, M), lambda c: (c, 0)),
            pl.BlockSpec((H, ROWS), lambda c: (0, 0)),
        ],
        out_specs=pl.BlockSpec((H, IB, M), lambda c: (0, c, 0)),
        out_shape=jax.ShapeDtypeStruct((H, M, M), jnp.float32),
    )(relative_position_index, bias_table.T)

    att3 = att_scores.reshape(W, SL, 128)
    bias2 = bias.reshape(SL, 128)
    out3 = pl.pallas_call(
        _add_body,
        grid=(NSTEP,),
        in_specs=[
            pl.BlockSpec((SL, 128), lambda i: (0, 0)),
            pl.BlockSpec(memory_space=pl.ANY),
        ],
        out_specs=pl.BlockSpec(memory_space=pl.ANY),
        out_shape=jax.ShapeDtypeStruct((W, SL, 128), jnp.float32),
        scratch_shapes=[
            pltpu.VMEM((2, NB, SL, 128), jnp.float32),
            pltpu.VMEM((2, NB, SL, 128), jnp.float32),
            pltpu.SemaphoreType.DMA((2,)),
            pltpu.SemaphoreType.DMA((2,)),
        ],
    )(bias2, att3)
    return out3.reshape(W, H, M, M)


# speedup vs baseline: 1.0001x; 1.0001x over previous
import jax
import jax.numpy as jnp
from jax.experimental import pallas as pl
from jax.experimental.pallas import tpu as pltpu

W = 256
H = 16
M = 144
ROWS = 529
IB = 8
SL = H * M * M // 128
NB = 8
NSTEP = W // NB


def _gather_body(rpi_ref, btT_ref, out_ref):
    iota = jax.lax.broadcasted_iota(jnp.int32, (ROWS, M), 0)
    btT = btT_ref[...]
    for rr in range(IB):
        onehot = (rpi_ref[rr:rr + 1, :] == iota).astype(jnp.float32)
        out_ref[:, rr, :] = jnp.dot(btT, onehot,
                                    preferred_element_type=jnp.float32)


def _add_body(bias_ref, att_hbm, out_hbm, in_buf, out_buf, in_sem, out_sem):
    i = pl.program_id(0)
    slot = jax.lax.rem(i, 2)
    nslot = 1 - slot

    def cp_in(step, sl, k):
        return pltpu.make_async_copy(
            att_hbm.at[pl.ds(step * NB + k, 1)],
            in_buf.at[sl].at[pl.ds(k, 1)], in_sem.at[sl])

    def cp_out(step, sl, k):
        return pltpu.make_async_copy(
            out_buf.at[sl].at[pl.ds(k, 1)],
            out_hbm.at[pl.ds(step * NB + k, 1)], out_sem.at[sl])

    @pl.when(i == 0)
    def _():
        for k in range(NB):
            cp_in(i, slot, k).start()

    @pl.when(i + 1 < NSTEP)
    def _():
        for k in range(NB):
            cp_in(i + 1, nslot, k).start()

    for k in range(NB):
        cp_in(i, slot, k).wait()

    @pl.when(i >= 2)
    def _():
        for k in range(NB):
            cp_out(i - 2, slot, k).wait()

    out_buf[slot] = in_buf[slot] + bias_ref[...][None]
    for k in range(NB):
        cp_out(i, slot, k).start()

    @pl.when(i == NSTEP - 1)
    def _():
        for k in range(NB):
            cp_out(i - 1, nslot, k).wait()
            cp_out(i, slot, k).wait()


def kernel(att_scores, bias_table, relative_position_index):
    bias = pl.pallas_call(
        _gather_body,
        grid=(M // IB,),
        in_specs=[
            pl.BlockSpec((IB, M), lambda c: (c, 0)),
            pl.BlockSpec((H, ROWS), lambda c: (0, 0)),
        ],
        out_specs=pl.BlockSpec((H, IB, M), lambda c: (0, c, 0)),
        out_shape=jax.ShapeDtypeStruct((H, M, M), jnp.float32),
    )(relative_position_index, bias_table.T)

    att3 = att_scores.reshape(W, SL, 128)
    bias2 = bias.reshape(SL, 128)
    out3 = pl.pallas_call(
        _add_body,
        grid=(NSTEP,),
        in_specs=[
            pl.BlockSpec((SL, 128), lambda i: (0, 0)),
            pl.BlockSpec(memory_space=pl.ANY),
        ],
        out_specs=pl.BlockSpec(memory_space=pl.ANY),
        out_shape=jax.ShapeDtypeStruct((W, SL, 128), jnp.float32),
        scratch_shapes=[
            pltpu.VMEM((2, NB, SL, 128), jnp.float32),
            pltpu.VMEM((2, NB, SL, 128), jnp.float32),
            pltpu.SemaphoreType.DMA((2,)),
            pltpu.SemaphoreType.DMA((2,)),
        ],
    )(bias2, att3)
    return out3.reshape(W, H, M, M)


# speedup vs baseline: 1.0019x; 1.0018x over previous
import jax
import jax.numpy as jnp
from jax.experimental import pallas as pl
from jax.experimental.pallas import tpu as pltpu

W = 256
H = 16
M = 144
ROWS = 529
IB = 8
SL = H * M * M // 128
NB = 8
NSTEP = W // NB


def _gather_body(rpi_ref, btT_ref, out_ref):
    iota = jax.lax.broadcasted_iota(jnp.int32, (ROWS, M), 0)
    btT = btT_ref[...]
    for rr in range(IB):
        onehot = (rpi_ref[rr:rr + 1, :] == iota).astype(jnp.float32)
        out_ref[:, rr, :] = jnp.dot(btT, onehot,
                                    preferred_element_type=jnp.float32)


def _add_body(bias_ref, att_hbm, out_hbm, in_buf, out_buf, in_sem, out_sem):
    i = pl.program_id(0)
    slot = jax.lax.rem(i, 2)
    nslot = 1 - slot

    def cp_in(step, sl, k):
        return pltpu.async_copy(
            att_hbm.at[pl.ds(step * NB + k, 1)],
            in_buf.at[sl].at[pl.ds(k, 1)], in_sem.at[sl],
            priority=k % 2)

    def wt_in(sl, k):
        pltpu.make_async_copy(
            att_hbm.at[pl.ds(k, 1)],
            in_buf.at[sl].at[pl.ds(k, 1)], in_sem.at[sl]).wait()

    def cp_out(step, sl, k):
        return pltpu.async_copy(
            out_buf.at[sl].at[pl.ds(k, 1)],
            out_hbm.at[pl.ds(step * NB + k, 1)], out_sem.at[sl],
            priority=k % 2)

    def wt_out(sl, k):
        pltpu.make_async_copy(
            out_buf.at[sl].at[pl.ds(k, 1)],
            out_hbm.at[pl.ds(k, 1)], out_sem.at[sl]).wait()

    @pl.when(i == 0)
    def _():
        for k in range(NB):
            cp_in(i, slot, k)

    @pl.when(i + 1 < NSTEP)
    def _():
        for k in range(NB):
            cp_in(i + 1, nslot, k)

    for k in range(NB):
        wt_in(slot, k)

    @pl.when(i >= 2)
    def _():
        for k in range(NB):
            wt_out(slot, k)

    out_buf[slot] = in_buf[slot] + bias_ref[...][None]
    for k in range(NB):
        cp_out(i, slot, k)

    @pl.when(i == NSTEP - 1)
    def _():
        for k in range(NB):
            wt_out(nslot, k)
            wt_out(slot, k)


def kernel(att_scores, bias_table, relative_position_index):
    bias = pl.pallas_call(
        _gather_body,
        grid=(M // IB,),
        in_specs=[
            pl.BlockSpec((IB, M), lambda c: (c, 0)),
            pl.BlockSpec((H, ROWS), lambda c: (0, 0)),
        ],
        out_specs=pl.BlockSpec((H, IB, M), lambda c: (0, c, 0)),
        out_shape=jax.ShapeDtypeStruct((H, M, M), jnp.float32),
    )(relative_position_index, bias_table.T)

    att3 = att_scores.reshape(W, SL, 128)
    bias2 = bias.reshape(SL, 128)
    out3 = pl.pallas_call(
        _add_body,
        grid=(NSTEP,),
        in_specs=[
            pl.BlockSpec((SL, 128), lambda i: (0, 0)),
            pl.BlockSpec(memory_space=pl.ANY),
        ],
        out_specs=pl.BlockSpec(memory_space=pl.ANY),
        out_shape=jax.ShapeDtypeStruct((W, SL, 128), jnp.float32),
        scratch_shapes=[
            pltpu.VMEM((2, NB, SL, 128), jnp.float32),
            pltpu.VMEM((2, NB, SL, 128), jnp.float32),
            pltpu.SemaphoreType.DMA((2,)),
            pltpu.SemaphoreType.DMA((2,)),
        ],
    )(bias2, att3)
    return out3.reshape(W, H, M, M)
